# baseline (speedup 1.0000x reference)
import jax
import jax.numpy as jnp
from jax.experimental import pallas as pl


def _add_body(x_ref, p_ref, o_ref):
    o_ref[...] = x_ref[...] + p_ref[...]


def kernel(x, pos_table):
    B, P, D = x.shape
    F = P * D
    R, C = F // 128, 128
    x3 = x.reshape(B, R, C)
    p3 = pos_table.reshape(R, C)
    out = pl.pallas_call(
        _add_body,
        grid=(B,),
        in_specs=[
            pl.BlockSpec((1, R, C), lambda i: (i, 0, 0)),
            pl.BlockSpec((R, C), lambda i: (0, 0)),
        ],
        out_specs=pl.BlockSpec((1, R, C), lambda i: (i, 0, 0)),
        out_shape=jax.ShapeDtypeStruct((B, R, C), x.dtype),
    )(x3, p3)
    return out.reshape(B, P, D)


# speedup vs baseline: 1.1681x; 1.1681x over previous
import jax
import jax.numpy as jnp
from jax.experimental import pallas as pl
from jax.experimental.pallas import tpu as pltpu

_NBUF = 8


def _add_body(x_hbm, p_ref, o_hbm, ibuf, obuf, isems, osems):
    B = x_hbm.shape[0]
    pos = p_ref[...]

    def in_cp(i, s):
        return pltpu.make_async_copy(x_hbm.at[i], ibuf.at[s], isems.at[s])

    def out_cp(i, s):
        return pltpu.make_async_copy(obuf.at[s], o_hbm.at[i], osems.at[s])

    for s in range(min(_NBUF, B)):
        in_cp(s, s).start()
    for i in range(B):
        s = i % _NBUF
        in_cp(i, s).wait()
        if i >= _NBUF:
            out_cp(i - _NBUF, s).wait()
        obuf[s] = ibuf[s] + pos
        out_cp(i, s).start()
        if i + _NBUF < B:
            in_cp(i + _NBUF, s).start()
    for i in range(max(0, B - _NBUF), B):
        out_cp(i, i % _NBUF).wait()


def kernel(x, pos_table):
    B, P, D = x.shape
    F = P * D
    R, C = F // 128, 128
    x3 = x.reshape(B, R, C)
    p3 = pos_table.reshape(R, C)
    out = pl.pallas_call(
        _add_body,
        in_specs=[
            pl.BlockSpec(memory_space=pl.ANY),
            pl.BlockSpec(memory_space=pltpu.MemorySpace.VMEM),
        ],
        out_specs=pl.BlockSpec(memory_space=pl.ANY),
        out_shape=jax.ShapeDtypeStruct((B, R, C), x.dtype),
        scratch_shapes=[
            pltpu.VMEM((_NBUF, R, C), jnp.float32),
            pltpu.VMEM((_NBUF, R, C), jnp.float32),
            pltpu.SemaphoreType.DMA((_NBUF,)),
            pltpu.SemaphoreType.DMA((_NBUF,)),
        ],
    )(x3, p3)
    return out.reshape(B, P, D)


# speedup vs baseline: 2.1572x; 1.8468x over previous
import jax
import jax.numpy as jnp
from jax.experimental import pallas as pl
from jax.experimental.pallas import tpu as pltpu

_NBUF = 8


def _add_body(x_hbm, p_ref, o_hbm, ibuf, obuf, isems, osems):
    B = x_hbm.shape[0]
    pos = p_ref[...]

    def in_cp(i, s):
        return pltpu.make_async_copy(x_hbm.at[i], ibuf.at[s], isems.at[s])

    def out_cp(i, s):
        return pltpu.make_async_copy(obuf.at[s], o_hbm.at[i], osems.at[s])

    for s in range(min(_NBUF, B)):
        in_cp(s, s).start()
    for i in range(B):
        s = i % _NBUF
        in_cp(i, s).wait()
        if i >= _NBUF:
            out_cp(i - _NBUF, s).wait()
        obuf[s] = ibuf[s] + pos
        out_cp(i, s).start()
        if i + _NBUF < B:
            in_cp(i + _NBUF, s).start()
    for i in range(max(0, B - _NBUF), B):
        out_cp(i, i % _NBUF).wait()


def kernel(x, pos_table):
    B, P, D = x.shape
    out = pl.pallas_call(
        _add_body,
        in_specs=[
            pl.BlockSpec(memory_space=pl.ANY),
            pl.BlockSpec(memory_space=pltpu.MemorySpace.VMEM),
        ],
        out_specs=pl.BlockSpec(memory_space=pl.ANY),
        out_shape=jax.ShapeDtypeStruct((B, P, D), x.dtype),
        scratch_shapes=[
            pltpu.VMEM((_NBUF, P, D), jnp.float32),
            pltpu.VMEM((_NBUF, P, D), jnp.float32),
            pltpu.SemaphoreType.DMA((_NBUF,)),
            pltpu.SemaphoreType.DMA((_NBUF,)),
        ],
    )(x, pos_table)
    return out


# speedup vs baseline: 2.1703x; 1.0060x over previous
import jax
import jax.numpy as jnp
from jax.experimental import pallas as pl
from jax.experimental.pallas import tpu as pltpu

_NBUF = 16


def _add_body(x_hbm, p_ref, o_hbm, ibuf, obuf, isems, osems):
    B = x_hbm.shape[0]
    pos = p_ref[...]

    def in_cp(i, s):
        return pltpu.make_async_copy(x_hbm.at[i], ibuf.at[s], isems.at[s])

    def out_cp(i, s):
        return pltpu.make_async_copy(obuf.at[s], o_hbm.at[i], osems.at[s])

    for s in range(min(_NBUF, B)):
        in_cp(s, s).start()
    for i in range(B):
        s = i % _NBUF
        in_cp(i, s).wait()
        if i >= _NBUF:
            out_cp(i - _NBUF, s).wait()
        obuf[s] = ibuf[s] + pos
        out_cp(i, s).start()
        if i + _NBUF < B:
            in_cp(i + _NBUF, s).start()
    for i in range(max(0, B - _NBUF), B):
        out_cp(i, i % _NBUF).wait()


def kernel(x, pos_table):
    B, P, D = x.shape
    out = pl.pallas_call(
        _add_body,
        in_specs=[
            pl.BlockSpec(memory_space=pl.ANY),
            pl.BlockSpec(memory_space=pltpu.MemorySpace.VMEM),
        ],
        out_specs=pl.BlockSpec(memory_space=pl.ANY),
        out_shape=jax.ShapeDtypeStruct((B, P, D), x.dtype),
        scratch_shapes=[
            pltpu.VMEM((_NBUF, P, D), jnp.float32),
            pltpu.VMEM((_NBUF, P, D), jnp.float32),
            pltpu.SemaphoreType.DMA((_NBUF,)),
            pltpu.SemaphoreType.DMA((_NBUF,)),
        ],
    )(x, pos_table)
    return out
